# baseline (speedup 1.0000x reference)
import functools

import jax
import jax.numpy as jnp
from jax import lax
from jax.experimental import pallas as pl
from jax.experimental.pallas import tpu as pltpu
from jax.experimental.pallas import tpu_sc as plsc

BATCH = 4096
SEQ = 200
DIM = 64
PDIM = 128

NC = 2
NS = 16
NW = NC * NS
ROWS_PER_W = BATCH // NW
IDX_PER_W = ROWS_PER_W * SEQ
INV_SEQ = 1.0 / SEQ


def _build_kernel():
    mesh = plsc.VectorSubcoreMesh(core_axis_name="c", subcore_axis_name="s")

    @functools.partial(
        pl.kernel,
        out_type=jax.ShapeDtypeStruct((BATCH, DIM), jnp.float32),
        mesh=mesh,
        compiler_params=pltpu.CompilerParams(use_tc_tiling_on_sc=True),
        scratch_types=[
            pltpu.VMEM((IDX_PER_W,), jnp.int32),
            pltpu.VMEM((2, SEQ, PDIM), jnp.float32),
            pltpu.VMEM((ROWS_PER_W, DIM), jnp.float32),
            pltpu.SemaphoreType.DMA,
            pltpu.SemaphoreType.DMA,
        ],
    )
    def enc(ids_hbm, table_hbm, out_hbm, idx_v, rows_v, out_v, sem0, sem1):
        sems = (sem0, sem1)
        wid = lax.axis_index("s") * NC + lax.axis_index("c")

        pltpu.sync_copy(ids_hbm.at[pl.ds(wid * IDX_PER_W, IDX_PER_W)], idx_v)

        def fire(r, b):
            base = r * SEQ
            for off, n in ((0, 128), (128, SEQ - 128)):
                pltpu.async_copy(
                    table_hbm.at[idx_v.at[pl.ds(base + off, n)]],
                    rows_v.at[b, pl.ds(off, n)],
                    sems[b],
                )

        def drain(b):
            pltpu.make_async_copy(
                table_hbm.at[pl.ds(0, SEQ)], rows_v.at[b], sems[b]
            ).wait()

        def accum(r, b):
            def body(j, accs):
                a0, a1, a2, a3 = accs
                a0 = a0 + rows_v[b, j, pl.ds(0, 16)]
                a1 = a1 + rows_v[b, j, pl.ds(16, 16)]
                a2 = a2 + rows_v[b, j, pl.ds(32, 16)]
                a3 = a3 + rows_v[b, j, pl.ds(48, 16)]
                return a0, a1, a2, a3

            z = jnp.zeros((16,), jnp.float32)
            a0, a1, a2, a3 = lax.fori_loop(0, SEQ, body, (z, z, z, z))
            out_v[r, pl.ds(0, 16)] = a0 * INV_SEQ
            out_v[r, pl.ds(16, 16)] = a1 * INV_SEQ
            out_v[r, pl.ds(32, 16)] = a2 * INV_SEQ
            out_v[r, pl.ds(48, 16)] = a3 * INV_SEQ

        fire(0, 0)

        def outer(i, carry):
            r = i * 2
            fire(r + 1, 1)
            drain(0)
            accum(r, 0)

            @pl.when(r + 2 < ROWS_PER_W)
            def _():
                fire(r + 2, 0)

            drain(1)
            accum(r + 1, 1)
            return carry

        lax.fori_loop(0, ROWS_PER_W // 2, outer, 0)

        pltpu.sync_copy(out_v, out_hbm.at[pl.ds(wid * ROWS_PER_W, ROWS_PER_W)])

    return enc


_enc = _build_kernel()


def kernel(text_ids, table):
    ids_flat = text_ids.reshape(-1).astype(jnp.int32)
    table_p = jnp.pad(table, ((0, 0), (0, PDIM - DIM)))
    return _enc(ids_flat, table_p)
